# baseline (speedup 1.0000x reference)
import functools

import jax
import jax.numpy as jnp
from jax import lax
from jax.experimental import pallas as pl
from jax.experimental.pallas import tpu as pltpu
from jax.experimental.pallas import tpu_sc as plsc

GENE = 10000
B = 4096
D1, D2, D3 = 1024, 512, 256
K = 8192
COM_COST = 0.25

BM = 1024
N_BT = B // BM
BMD = 512
N_BTD = B // BMD
BMQ = 256
N_BTQ = B // BMQ
BK = 1000
NK = GENE // BK
BN = 2000
NJ = GENE // BN


def _enc_body(xt_ref, w1_ref, b1_ref, w2_ref, b2_ref, w3_ref, b3_ref,
              z_ref, acc_ref):
    k = pl.program_id(0)
    i = pl.program_id(1)
    xt = xt_ref[...]
    w1 = w1_ref[...]
    dn = (((0,), (0,)), ((), ()))
    part = lax.dot_general(xt, w1, dn, preferred_element_type=jnp.float32)
    base = i * BM

    @pl.when(k == 0)
    def _():
        acc_ref[pl.ds(base, BM), :] = part

    @pl.when(k > 0)
    def _():
        acc_ref[pl.ds(base, BM), :] += part

    @pl.when(k == NK - 1)
    def _():
        z1 = jnp.maximum(acc_ref[pl.ds(base, BM), :] + b1_ref[...], 0.0)
        z2 = jnp.maximum(
            jnp.dot(z1, w2_ref[...], preferred_element_type=jnp.float32)
            + b2_ref[...], 0.0)
        z3 = jnp.maximum(
            jnp.dot(z2, w3_ref[...], preferred_element_type=jnp.float32)
            + b3_ref[...], 0.0)
        z_ref[...] = z3


def _vq_body(z_ref, c_ref, idx_ref):
    z = z_ref[...]
    c = c_ref[...]
    a = jnp.sum(z * z, axis=1, keepdims=True)
    bb = jnp.sum(c * c, axis=1)[None, :]
    zc = lax.dot_general(z, c, (((1,), (1,)), ((), ())),
                         preferred_element_type=jnp.float32)
    d = (a + bb) - 2.0 * zc
    m = jnp.min(d, axis=1, keepdims=True)
    ks = lax.broadcasted_iota(jnp.int32, d.shape, 1)
    idx = jnp.min(jnp.where(d == m, ks, K), axis=1).astype(jnp.int32)
    idx_ref[...] = idx


_NC, _NS = 2, 16
_NW = _NC * _NS
_BPW = B // _NW


def _make_sc_gather():
    @functools.partial(
        pl.kernel,
        mesh=plsc.VectorSubcoreMesh(core_axis_name="c", subcore_axis_name="s"),
        out_type=jax.ShapeDtypeStruct((B, D3), jnp.float32),
        scratch_types=[
            pltpu.VMEM((_BPW,), jnp.int32),
            pltpu.VMEM((_BPW, D3), jnp.float32),
            pltpu.SemaphoreType.DMA,
        ],
    )
    def _sc_gather(table_hbm, idx_hbm, out_hbm, idx_v, rows_v, sem):
        wid = lax.axis_index("s") * _NC + lax.axis_index("c")
        base = wid * _BPW
        pltpu.sync_copy(idx_hbm.at[pl.ds(base, _BPW)], idx_v)
        pltpu.async_copy(table_hbm.at[idx_v], rows_v, sem).wait()
        pltpu.sync_copy(rows_v, out_hbm.at[pl.ds(base, _BPW)])

    return _sc_gather


def _dec_body(z_ref, q_ref, w4_ref, b4_ref, w5_ref, b5_ref, w6t_ref, b6_ref,
              qst_ref, xrect_ref, ls_ref, h_ref, w6s_ref):
    j = pl.program_id(0)
    i = pl.program_id(1)
    z = z_ref[...]
    q = q_ref[...]
    qst = z + (q - z)
    qst_ref[...] = qst

    @pl.when(i == 0)
    def _():
        w6s_ref[...] = w6t_ref[...].astype(jnp.bfloat16)

    @pl.when(jnp.logical_and(j == 0, i == 0))
    def _():
        ls_ref[...] = jnp.zeros((1, 1), jnp.float32)

    @pl.when(j == 0)
    def _():
        diff = q - z
        ls_ref[...] += jnp.sum(diff * diff, axis=(0, 1), keepdims=True)
        h1 = jnp.maximum(
            jnp.dot(qst.astype(jnp.bfloat16), w4_ref[...],
                    preferred_element_type=jnp.float32)
            + b4_ref[...], 0.0)
        h2 = jnp.maximum(
            jnp.dot(h1.astype(jnp.bfloat16), w5_ref[...],
                    preferred_element_type=jnp.float32)
            + b5_ref[...], 0.0)
        h_ref[pl.ds(i * BMD, BMD), :] = h2.astype(jnp.bfloat16)

    h = h_ref[pl.ds(i * BMD, BMD), :]
    w6t = w6s_ref[...]
    xrect = lax.dot_general(w6t, h, (((1,), (1,)), ((), ())),
                            preferred_element_type=jnp.float32)
    xrect_ref[...] = xrect + b6_ref[...]


def kernel(inputs, W1, b1, W2, b2, W3, b3, codebook, W4, b4, W5, b5, W6, b6):
    b1r = b1.reshape(1, D1)
    b2r = b2.reshape(1, D2)
    b3r = b3.reshape(1, D3)
    b4r = b4.reshape(1, D2)
    b5r = b5.reshape(1, D1)
    b6r = b6.reshape(GENE, 1)

    xt = inputs.T
    w4b = W4.astype(jnp.bfloat16)
    w5b = W5.astype(jnp.bfloat16)
    w6t = W6.T

    z = pl.pallas_call(
        _enc_body,
        grid=(NK, N_BT),
        in_specs=[
            pl.BlockSpec((BK, BM), lambda k, i: (k, i)),
            pl.BlockSpec((BK, D1), lambda k, i: (k, 0)),
            pl.BlockSpec((1, D1), lambda k, i: (0, 0)),
            pl.BlockSpec((D1, D2), lambda k, i: (0, 0)),
            pl.BlockSpec((1, D2), lambda k, i: (0, 0)),
            pl.BlockSpec((D2, D3), lambda k, i: (0, 0)),
            pl.BlockSpec((1, D3), lambda k, i: (0, 0)),
        ],
        out_specs=pl.BlockSpec((BM, D3), lambda k, i: (i, 0)),
        out_shape=jax.ShapeDtypeStruct((B, D3), jnp.float32),
        scratch_shapes=[pltpu.VMEM((B, D1), jnp.float32)],
    )(xt, W1, b1r, W2, b2r, W3, b3r)

    idx3 = pl.pallas_call(
        _vq_body,
        grid=(N_BTQ,),
        in_specs=[
            pl.BlockSpec((BMQ, D3), lambda i: (i, 0)),
            pl.BlockSpec((K, D3), lambda i: (0, 0)),
        ],
        out_specs=pl.BlockSpec((BMQ,), lambda i: (i,)),
        out_shape=jax.ShapeDtypeStruct((B,), jnp.int32),
    )(z, codebook)

    quantized = _make_sc_gather()(codebook, idx3)

    qst, xrect, ls = pl.pallas_call(
        _dec_body,
        grid=(NJ, N_BTD),
        in_specs=[
            pl.BlockSpec((BMD, D3), lambda j, i: (i, 0)),
            pl.BlockSpec((BMD, D3), lambda j, i: (i, 0)),
            pl.BlockSpec((D3, D2), lambda j, i: (0, 0)),
            pl.BlockSpec((1, D2), lambda j, i: (0, 0)),
            pl.BlockSpec((D2, D1), lambda j, i: (0, 0)),
            pl.BlockSpec((1, D1), lambda j, i: (0, 0)),
            pl.BlockSpec((BN, D1), lambda j, i: (j, 0)),
            pl.BlockSpec((BN, 1), lambda j, i: (j, 0)),
        ],
        out_specs=[
            pl.BlockSpec((BMD, D3), lambda j, i: (i, 0)),
            pl.BlockSpec((BN, BMD), lambda j, i: (j, i)),
            pl.BlockSpec((1, 1), lambda j, i: (0, 0)),
        ],
        out_shape=[
            jax.ShapeDtypeStruct((B, D3), jnp.float32),
            jax.ShapeDtypeStruct((GENE, B), jnp.float32),
            jax.ShapeDtypeStruct((1, 1), jnp.float32),
        ],
        scratch_shapes=[pltpu.VMEM((B, D1), jnp.bfloat16),
                        pltpu.VMEM((BN, D1), jnp.bfloat16)],
    )(z, quantized, w4b, b4r, w5b, b5r, w6t, b6r)

    xrec = xrect.T
    mean_se = ls[0, 0] / (B * D3)
    loss = mean_se + COM_COST * mean_se
    return (loss, xrec, qst)


# speedup vs baseline: 1.0352x; 1.0352x over previous
import functools

import jax
import jax.numpy as jnp
from jax import lax
from jax.experimental import pallas as pl
from jax.experimental.pallas import tpu as pltpu
from jax.experimental.pallas import tpu_sc as plsc

GENE = 10000
B = 4096
D1, D2, D3 = 1024, 512, 256
K = 8192
COM_COST = 0.25

BM = 512
N_BT = B // BM
BMQ = 256
N_BTQ = B // BMQ
BK = 2000
NK = GENE // BK
BN = 2000
NJ = GENE // BN


def _enc_body(xt_ref, w1_ref, b1_ref, w2_ref, b2_ref, w3_ref, b3_ref,
              z_ref, acc_ref):
    k = pl.program_id(0)
    i = pl.program_id(1)
    xt = xt_ref[...]
    w1 = w1_ref[...]
    dn = (((0,), (0,)), ((), ()))
    part = lax.dot_general(xt, w1, dn, preferred_element_type=jnp.float32)
    base = i * BM

    @pl.when(k == 0)
    def _():
        acc_ref[pl.ds(base, BM), :] = part

    @pl.when(k > 0)
    def _():
        acc_ref[pl.ds(base, BM), :] += part

    @pl.when(k == NK - 1)
    def _():
        z1 = jnp.maximum(acc_ref[pl.ds(base, BM), :] + b1_ref[...], 0.0)
        z2 = jnp.maximum(
            jnp.dot(z1, w2_ref[...], preferred_element_type=jnp.float32)
            + b2_ref[...], 0.0)
        z3 = jnp.maximum(
            jnp.dot(z2, w3_ref[...], preferred_element_type=jnp.float32)
            + b3_ref[...], 0.0)
        z_ref[...] = z3


def _vq_body(z_ref, c_ref, idx_ref):
    z = z_ref[...]
    c = c_ref[...]
    a = jnp.sum(z * z, axis=1, keepdims=True)
    bb = jnp.sum(c * c, axis=1)[None, :]
    zc = lax.dot_general(z, c, (((1,), (1,)), ((), ())),
                         preferred_element_type=jnp.float32)
    d = (a + bb) - 2.0 * zc
    m = jnp.min(d, axis=1, keepdims=True)
    ks = lax.broadcasted_iota(jnp.int32, d.shape, 1)
    idx = jnp.min(jnp.where(d == m, ks, K), axis=1).astype(jnp.int32)
    idx_ref[...] = idx


_NC, _NS = 2, 16
_NW = _NC * _NS
BH = B // 2
_BPW = BH // _NW


def _make_sc_gather():
    @functools.partial(
        pl.kernel,
        mesh=plsc.VectorSubcoreMesh(core_axis_name="c", subcore_axis_name="s"),
        out_type=jax.ShapeDtypeStruct((BH, D3), jnp.float32),
        scratch_types=[
            pltpu.VMEM((_BPW,), jnp.int32),
            pltpu.VMEM((_BPW, D3), jnp.float32),
            pltpu.SemaphoreType.DMA,
        ],
    )
    def _sc_gather(table_hbm, idx_hbm, out_hbm, idx_v, rows_v, sem):
        wid = lax.axis_index("s") * _NC + lax.axis_index("c")
        base = wid * _BPW
        pltpu.sync_copy(idx_hbm.at[pl.ds(base, _BPW)], idx_v)
        pltpu.async_copy(table_hbm.at[idx_v], rows_v, sem).wait()
        pltpu.sync_copy(rows_v, out_hbm.at[pl.ds(base, _BPW)])

    return _sc_gather


def _dec_body(z_ref, q_ref, w4_ref, b4_ref, w5_ref, b5_ref, w6t_ref, b6_ref,
              qst_ref, xrect_ref, ls_ref, h_ref, w6s_ref):
    j = pl.program_id(0)
    i = pl.program_id(1)
    z = z_ref[...]
    q = q_ref[...]
    qst = z + (q - z)
    qst_ref[...] = qst

    @pl.when(i == 0)
    def _():
        w6s_ref[...] = w6t_ref[...].astype(jnp.bfloat16)

    @pl.when(jnp.logical_and(j == 0, i == 0))
    def _():
        ls_ref[...] = jnp.zeros((1, 1), jnp.float32)

    @pl.when(j == 0)
    def _():
        diff = q - z
        ls_ref[...] += jnp.sum(diff * diff, axis=(0, 1), keepdims=True)
        h1 = jnp.maximum(
            jnp.dot(qst.astype(jnp.bfloat16), w4_ref[...],
                    preferred_element_type=jnp.float32)
            + b4_ref[...], 0.0)
        h2 = jnp.maximum(
            jnp.dot(h1.astype(jnp.bfloat16), w5_ref[...],
                    preferred_element_type=jnp.float32)
            + b5_ref[...], 0.0)
        h_ref[pl.ds(i * BM, BM), :] = h2.astype(jnp.bfloat16)

    h = h_ref[pl.ds(i * BM, BM), :]
    w6t = w6s_ref[...]
    xrect = lax.dot_general(w6t, h, (((1,), (1,)), ((), ())),
                            preferred_element_type=jnp.float32)
    xrect_ref[...] = xrect + b6_ref[...]


def kernel(inputs, W1, b1, W2, b2, W3, b3, codebook, W4, b4, W5, b5, W6, b6):
    b1r = b1.reshape(1, D1)
    b2r = b2.reshape(1, D2)
    b3r = b3.reshape(1, D3)
    b4r = b4.reshape(1, D2)
    b5r = b5.reshape(1, D1)
    b6r = b6.reshape(GENE, 1)

    xt = inputs.T
    w4b = W4.astype(jnp.bfloat16)
    w5b = W5.astype(jnp.bfloat16)
    w6t = W6.T

    z = pl.pallas_call(
        _enc_body,
        grid=(NK, N_BT),
        in_specs=[
            pl.BlockSpec((BK, BM), lambda k, i: (k, i)),
            pl.BlockSpec((BK, D1), lambda k, i: (k, 0)),
            pl.BlockSpec((1, D1), lambda k, i: (0, 0)),
            pl.BlockSpec((D1, D2), lambda k, i: (0, 0)),
            pl.BlockSpec((1, D2), lambda k, i: (0, 0)),
            pl.BlockSpec((D2, D3), lambda k, i: (0, 0)),
            pl.BlockSpec((1, D3), lambda k, i: (0, 0)),
        ],
        out_specs=pl.BlockSpec((BM, D3), lambda k, i: (i, 0)),
        out_shape=jax.ShapeDtypeStruct((B, D3), jnp.float32),
        scratch_shapes=[pltpu.VMEM((B, D1), jnp.float32)],
    )(xt, W1, b1r, W2, b2r, W3, b3r)

    gather = _make_sc_gather()
    nh = BH // BMQ
    qhalves = []
    for h in range(2):
        idxh = pl.pallas_call(
            _vq_body,
            grid=(nh,),
            in_specs=[
                pl.BlockSpec((BMQ, D3), lambda i, h=h: (h * nh + i, 0)),
                pl.BlockSpec((K, D3), lambda i: (0, 0)),
            ],
            out_specs=pl.BlockSpec((BMQ,), lambda i: (i,)),
            out_shape=jax.ShapeDtypeStruct((BH,), jnp.int32),
        )(z, codebook)
        qhalves.append(gather(codebook, idxh))
    quantized = jnp.concatenate(qhalves, axis=0)

    qst, xrect, ls = pl.pallas_call(
        _dec_body,
        grid=(NJ, N_BT),
        in_specs=[
            pl.BlockSpec((BM, D3), lambda j, i: (i, 0)),
            pl.BlockSpec((BM, D3), lambda j, i: (i, 0)),
            pl.BlockSpec((D3, D2), lambda j, i: (0, 0)),
            pl.BlockSpec((1, D2), lambda j, i: (0, 0)),
            pl.BlockSpec((D2, D1), lambda j, i: (0, 0)),
            pl.BlockSpec((1, D1), lambda j, i: (0, 0)),
            pl.BlockSpec((BN, D1), lambda j, i: (j, 0)),
            pl.BlockSpec((BN, 1), lambda j, i: (j, 0)),
        ],
        out_specs=[
            pl.BlockSpec((BM, D3), lambda j, i: (i, 0)),
            pl.BlockSpec((BN, BM), lambda j, i: (j, i)),
            pl.BlockSpec((1, 1), lambda j, i: (0, 0)),
        ],
        out_shape=[
            jax.ShapeDtypeStruct((B, D3), jnp.float32),
            jax.ShapeDtypeStruct((GENE, B), jnp.float32),
            jax.ShapeDtypeStruct((1, 1), jnp.float32),
        ],
        scratch_shapes=[pltpu.VMEM((B, D1), jnp.bfloat16),
                        pltpu.VMEM((BN, D1), jnp.bfloat16)],
    )(z, quantized, w4b, b4r, w5b, b5r, w6t, b6r)

    xrec = xrect.T
    mean_se = ls[0, 0] / (B * D3)
    loss = mean_se + COM_COST * mean_se
    return (loss, xrec, qst)
